# baseline (speedup 1.0000x reference)
import functools

import jax
import jax.numpy as jnp
from jax import lax
from jax.experimental import pallas as pl
from jax.experimental.pallas import tpu as pltpu
from jax.experimental.pallas import tpu_sc as plsc

NUM_EMB = 1000000
DIM = 32
N_LOOKUPS = 327680

_INFO = plsc.get_sparse_core_info()
NC = _INFO.num_cores
NS = _INFO.num_subcores
NW = NC * NS
LANES = _INFO.num_lanes

B_PER_W = N_LOOKUPS // NW
CHUNK = 2048
N_CHUNKS = B_PER_W // CHUNK


def _sc_lookup(idx0_hbm, idx1_hbm, t0_hbm, t1_hbm,
               out0_hbm, out1_hbm, parts_hbm,
               idx_v, rows_v, acc_v, sem):
    wid = lax.axis_index("s") * NC + lax.axis_index("c")
    base = wid * B_PER_W

    def do_table(idx_hbm, t_hbm, out_hbm, carry):
        def chunk_body(i, carry):
            off = base + i * CHUNK
            pltpu.sync_copy(idx_hbm.at[pl.ds(off, CHUNK)], idx_v)
            pltpu.async_copy(t_hbm.at[idx_v], rows_v, sem).wait()
            pltpu.sync_copy(rows_v, out_hbm.at[pl.ds(off, CHUNK)])

            def sum_body(j, c):
                a, b = c
                return (a + rows_v[j, pl.ds(0, LANES)],
                        b + rows_v[j, pl.ds(LANES, LANES)])

            return lax.fori_loop(0, CHUNK, sum_body, carry, unroll=8)

        return lax.fori_loop(0, N_CHUNKS, chunk_body, carry)

    zeros = jnp.zeros((LANES,), jnp.float32)
    carry = (zeros, zeros)
    carry = do_table(idx0_hbm, t0_hbm, out0_hbm, carry)
    carry = do_table(idx1_hbm, t1_hbm, out1_hbm, carry)
    acc_v[...] = carry[0] + carry[1]
    pltpu.sync_copy(acc_v, parts_hbm.at[wid])


@jax.jit
def kernel(indices_0, indices_1, table_0, table_1):
    mesh = plsc.VectorSubcoreMesh(core_axis_name="c", subcore_axis_name="s")
    call = functools.partial(
        pl.kernel,
        mesh=mesh,
        compiler_params=pltpu.CompilerParams(use_tc_tiling_on_sc=False),
        out_type=(
            jax.ShapeDtypeStruct((N_LOOKUPS, DIM), jnp.float32),
            jax.ShapeDtypeStruct((N_LOOKUPS, DIM), jnp.float32),
            jax.ShapeDtypeStruct((NW, LANES), jnp.float32),
        ),
        scratch_types=[
            pltpu.VMEM((CHUNK,), jnp.int32),
            pltpu.VMEM((CHUNK, DIM), jnp.float32),
            pltpu.VMEM((LANES,), jnp.float32),
            pltpu.SemaphoreType.DMA,
        ],
    )(_sc_lookup)
    emb_0, emb_1, parts = call(indices_0, indices_1, table_0, table_1)
    loss = jnp.sum(parts) / jnp.float32(2 * N_LOOKUPS * DIM)
    return (loss, emb_0, emb_1)


# speedup vs baseline: 1.0016x; 1.0016x over previous
import functools

import jax
import jax.numpy as jnp
from jax import lax
from jax.experimental import pallas as pl
from jax.experimental.pallas import tpu as pltpu
from jax.experimental.pallas import tpu_sc as plsc

NUM_EMB = 1000000
DIM = 32
N_LOOKUPS = 327680

_INFO = plsc.get_sparse_core_info()
NC = _INFO.num_cores
NS = _INFO.num_subcores
NW = NC * NS
LANES = _INFO.num_lanes

B_PER_W = N_LOOKUPS // NW
CHUNK = 2048
N_CHUNKS = B_PER_W // CHUNK


def _sc_lookup(idx0_hbm, idx1_hbm, t0_hbm, t1_hbm,
               out0_hbm, out1_hbm, parts_hbm,
               idx_v, rows_v, acc_v, sem):
    wid = lax.axis_index("s") * NC + lax.axis_index("c")
    base = wid * B_PER_W

    def do_table(idx_hbm, t_hbm, out_hbm, carry):
        def chunk_body(i, carry):
            off = base + i * CHUNK
            pltpu.sync_copy(idx_hbm.at[pl.ds(off, CHUNK)], idx_v)
            pltpu.async_copy(t_hbm.at[idx_v], rows_v, sem).wait()
            pltpu.sync_copy(rows_v, out_hbm.at[pl.ds(off, CHUNK)])

            def sum_body(j, c):
                a, b = c
                return (a + rows_v[j, pl.ds(0, LANES)],
                        b + rows_v[j, pl.ds(LANES, LANES)])

            return lax.fori_loop(0, CHUNK, sum_body, carry, unroll=8)

        return lax.fori_loop(0, N_CHUNKS, chunk_body, carry)

    zeros = jnp.zeros((LANES,), jnp.float32)
    carry = (zeros, zeros)
    carry = do_table(idx0_hbm, t0_hbm, out0_hbm, carry)
    carry = do_table(idx1_hbm, t1_hbm, out1_hbm, carry)
    acc_v[...] = carry[0] + carry[1]
    pltpu.sync_copy(acc_v, parts_hbm.at[wid])


@jax.jit
def kernel(indices_0, indices_1, table_0, table_1):
    mesh = plsc.VectorSubcoreMesh(core_axis_name="c", subcore_axis_name="s")
    call = functools.partial(
        pl.kernel,
        mesh=mesh,
        compiler_params=pltpu.CompilerParams(use_tc_tiling_on_sc=False),
        out_type=(
            jax.ShapeDtypeStruct((N_LOOKUPS, DIM), jnp.float32),
            jax.ShapeDtypeStruct((N_LOOKUPS, DIM), jnp.float32),
            jax.ShapeDtypeStruct((NW, LANES), jnp.float32),
        ),
        scratch_types=[
            pltpu.VMEM((CHUNK,), jnp.int32),
            pltpu.VMEM((CHUNK, DIM), jnp.float32),
            pltpu.VMEM((LANES,), jnp.float32),
            pltpu.SemaphoreType.DMA,
        ],
    )(_sc_lookup)
    t0 = table_0.reshape(-1).reshape(NUM_EMB, DIM)
    t1 = table_1.reshape(-1).reshape(NUM_EMB, DIM)
    emb_0, emb_1, parts = call(indices_0, indices_1, t0, t1)
    loss = jnp.sum(parts) / jnp.float32(2 * N_LOOKUPS * DIM)
    return (loss, emb_0, emb_1)
